# baseline (speedup 1.0000x reference)
import functools

import jax
import jax.numpy as jnp
from jax import lax
from jax.experimental import pallas as pl
from jax.experimental.pallas import tpu as pltpu
from jax.experimental.pallas import tpu_sc as plsc

B = 4096
L = 200
EMB = 64
N = B * L
NC = 2
NS = 16
NW = NC * NS
PER_W = N // NW
NCHUNK = PER_W // L
SPLIT = 128
NBUF = 2


def _body(ids_hbm, table_hbm, pos_hbm, out_hbm,
          idx_v, pos_v, gbuf, sbuf,
          gsem0, gsem1, ssem0, ssem1):
  gsems = (gsem0, gsem1)
  ssems = (ssem0, ssem1)
  cid = lax.axis_index("c")
  sid = lax.axis_index("s")
  wid = sid * NC + cid
  base = wid * PER_W

  pltpu.sync_copy(ids_hbm.at[pl.ds(base * 1, PER_W)], idx_v)
  pltpu.sync_copy(pos_hbm, pos_v)

  def gather_start(c, b):
    off = c * L
    pltpu.make_async_copy(
        table_hbm.at[idx_v.at[pl.ds(off, SPLIT)]],
        gbuf.at[b, pl.ds(0, SPLIT)], gsems[b]).start()
    pltpu.make_async_copy(
        table_hbm.at[idx_v.at[pl.ds(off + SPLIT, L - SPLIT)]],
        gbuf.at[b, pl.ds(SPLIT, L - SPLIT)], gsems[b]).start()

  def gather_wait(c, b):
    off = c * L
    pltpu.make_async_copy(
        table_hbm.at[idx_v.at[pl.ds(off, SPLIT)]],
        gbuf.at[b, pl.ds(0, SPLIT)], gsems[b]).wait()
    pltpu.make_async_copy(
        table_hbm.at[idx_v.at[pl.ds(off + SPLIT, L - SPLIT)]],
        gbuf.at[b, pl.ds(SPLIT, L - SPLIT)], gsems[b]).wait()

  def store_start(c, b):
    pltpu.make_async_copy(
        sbuf.at[b], out_hbm.at[pl.ds(base + c * L, L)], ssems[b]).start()

  def store_wait(c, b):
    pltpu.make_async_copy(
        sbuf.at[b], out_hbm.at[pl.ds(base + c * L, L)], ssems[b]).wait()

  def add_pos(b):
    def row(k, _):
      r = k >> 2
      col = (k & 3) * 16
      sbuf[b, r, pl.ds(col, 16)] = (
          gbuf[b, r, pl.ds(col, 16)] + pos_v[r, pl.ds(col, 16)])
      return _
    lax.fori_loop(0, L * 4, row, 0, unroll=4)

  for b in range(NBUF):
    gather_start(b, b)

  for b in range(NBUF):
    gather_wait(b, b)
    add_pos(b)
    store_start(b, b)
    gather_start(b + NBUF, b)

  def group(g, _):
    for b in range(NBUF):
      c = g * NBUF + b
      gather_wait(c, b)
      store_wait(c - NBUF, b)
      add_pos(b)
      store_start(c, b)
      gather_start(c + NBUF, b)
    return _
  lax.fori_loop(1, NCHUNK // NBUF - 1, group, 0)

  for b in range(NBUF):
    c = NCHUNK - NBUF + b
    gather_wait(c, b)
    store_wait(c - NBUF, b)
    add_pos(b)
    store_start(c, b)
  for b in range(NBUF):
    store_wait(NCHUNK - NBUF + b, b)


@jax.jit
def _run(ids_flat, token_table, pos_table):
  mesh = plsc.VectorSubcoreMesh(core_axis_name="c", subcore_axis_name="s")
  f = pl.kernel(
      _body,
      out_type=jax.ShapeDtypeStruct((N, EMB), jnp.float32),
      mesh=mesh,
      scratch_types=[
          pltpu.VMEM((PER_W,), jnp.int32),
          pltpu.VMEM((L, EMB), jnp.float32),
          pltpu.VMEM((NBUF, L, EMB), jnp.float32),
          pltpu.VMEM((NBUF, L, EMB), jnp.float32),
          pltpu.SemaphoreType.DMA,
          pltpu.SemaphoreType.DMA,
          pltpu.SemaphoreType.DMA,
          pltpu.SemaphoreType.DMA,
      ],
      compiler_params=pltpu.CompilerParams(use_tc_tiling_on_sc=False),
  )
  return f(ids_flat, token_table, pos_table)


def kernel(input_ids, token_table, pos_table):
  ids_flat = input_ids.reshape(-1).astype(jnp.int32)
  out = _run(ids_flat, token_table, pos_table)
  return out.reshape(B, L, EMB)


# speedup vs baseline: 1.1049x; 1.1049x over previous
import functools

import jax
import jax.numpy as jnp
from jax import lax
from jax.experimental import pallas as pl
from jax.experimental.pallas import tpu as pltpu
from jax.experimental.pallas import tpu_sc as plsc

B = 4096
L = 200
EMB = 64
N = B * L
NC = 2
NS = 16
NW = NC * NS
PER_W = N // NW
NCHUNK = PER_W // L
SPLIT = 128
NGBUF = 3
NSBUF = 2


def _body(ids_hbm, table_hbm, pos_hbm, out_hbm,
          idx_v, pos_v, gbuf, sbuf,
          gsem0, gsem1, gsem2, ssem0, ssem1):
  gsems = (gsem0, gsem1, gsem2)
  ssems = (ssem0, ssem1)
  cid = lax.axis_index("c")
  sid = lax.axis_index("s")
  wid = sid * NC + cid
  base = wid * PER_W

  pltpu.sync_copy(ids_hbm.at[pl.ds(base * 1, PER_W)], idx_v)
  pltpu.sync_copy(pos_hbm, pos_v)

  def gather_start(c, b):
    off = c * L
    pltpu.make_async_copy(
        table_hbm.at[idx_v.at[pl.ds(off, SPLIT)]],
        gbuf.at[b, pl.ds(0, SPLIT)], gsems[b]).start()
    pltpu.make_async_copy(
        table_hbm.at[idx_v.at[pl.ds(off + SPLIT, L - SPLIT)]],
        gbuf.at[b, pl.ds(SPLIT, L - SPLIT)], gsems[b]).start()

  def gather_wait(c, b):
    off = c * L
    pltpu.make_async_copy(
        table_hbm.at[idx_v.at[pl.ds(off, SPLIT)]],
        gbuf.at[b, pl.ds(0, SPLIT)], gsems[b]).wait()
    pltpu.make_async_copy(
        table_hbm.at[idx_v.at[pl.ds(off + SPLIT, L - SPLIT)]],
        gbuf.at[b, pl.ds(SPLIT, L - SPLIT)], gsems[b]).wait()

  def store_start(c, b):
    pltpu.make_async_copy(
        sbuf.at[b], out_hbm.at[pl.ds(base + c * L, L)], ssems[b]).start()

  def store_wait(c, b):
    pltpu.make_async_copy(
        sbuf.at[b], out_hbm.at[pl.ds(base + c * L, L)], ssems[b]).wait()

  def add_pos(gb, sb):
    def row(k, _):
      r = k >> 2
      col = (k & 3) * 16
      sbuf[sb, r, pl.ds(col, 16)] = (
          gbuf[gb, r, pl.ds(col, 16)] + pos_v[r, pl.ds(col, 16)])
      return _
    lax.fori_loop(0, L * 4, row, 0, unroll=8)

  def chunk_body(c, gb, sb, with_store_wait=True, with_gather=True):
    gather_wait(c, gb)
    if with_store_wait:
      store_wait(c - NSBUF, sb)
    add_pos(gb, sb)
    store_start(c, sb)
    if with_gather:
      gather_start(c + NGBUF, gb)

  for b in range(NGBUF):
    gather_start(b, b)

  for c in range(NSBUF):
    chunk_body(c, c % NGBUF, c % NSBUF, with_store_wait=False)
  for c in range(NSBUF, 6):
    chunk_body(c, c % NGBUF, c % NSBUF)

  def group(g, _):
    for i in range(6):
      c = g * 6 + i
      chunk_body(c, i % NGBUF, i % NSBUF)
    return _
  lax.fori_loop(1, 20, group, 0)

  for c in range(120, NCHUNK):
    chunk_body(c, c % NGBUF, c % NSBUF, with_gather=(c + NGBUF < NCHUNK))
  for c in range(NCHUNK - NSBUF, NCHUNK):
    store_wait(c, c % NSBUF)


@jax.jit
def _run(ids_flat, token_table, pos_table):
  mesh = plsc.VectorSubcoreMesh(core_axis_name="c", subcore_axis_name="s")
  f = pl.kernel(
      _body,
      out_type=jax.ShapeDtypeStruct((N, EMB), jnp.float32),
      mesh=mesh,
      scratch_types=[
          pltpu.VMEM((PER_W,), jnp.int32),
          pltpu.VMEM((L, EMB), jnp.float32),
          pltpu.VMEM((NGBUF, L, EMB), jnp.float32),
          pltpu.VMEM((NSBUF, L, EMB), jnp.float32),
          pltpu.SemaphoreType.DMA,
          pltpu.SemaphoreType.DMA,
          pltpu.SemaphoreType.DMA,
          pltpu.SemaphoreType.DMA,
          pltpu.SemaphoreType.DMA,
      ],
      compiler_params=pltpu.CompilerParams(
          use_tc_tiling_on_sc=False,
          disable_bounds_checks=True,
          disable_semaphore_checks=True,
      ),
  )
  return f(ids_flat, token_table, pos_table)


def kernel(input_ids, token_table, pos_table):
  ids_flat = input_ids.reshape(-1).astype(jnp.int32)
  out = _run(ids_flat, token_table, pos_table)
  return out.reshape(B, L, EMB)
